# baseline (speedup 1.0000x reference)
import functools

import jax
import jax.numpy as jnp
from jax.experimental import pallas as pl
from jax.experimental.pallas import tpu as pltpu

BLK = 128
TB = 256
E = 8
K = 2


def _router_shared_body(x16_ref, rw_ref, sg_ref, su_ref, sd_ref, seg_ref,
                        sh_ref, m_ref):
    xb = x16_ref[...]
    logits = jax.lax.dot_general(
        xb, rw_ref[...], (((1,), (1,)), ((), ())),
        preferred_element_type=jnp.float32)
    mx = jnp.max(logits, axis=1, keepdims=True)
    ex = jnp.exp(logits - mx)
    p = ex / jnp.sum(ex, axis=1, keepdims=True)
    lane = jax.lax.broadcasted_iota(jnp.int32, p.shape, 1).astype(jnp.float32)
    m1 = jnp.max(p, axis=1, keepdims=True)
    i1 = jnp.min(jnp.where(p == m1, lane, jnp.float32(1e9)), axis=1,
                 keepdims=True)
    p2 = jnp.where(lane == i1, jnp.float32(-1e30), p)
    m2 = jnp.max(p2, axis=1, keepdims=True)
    i2 = jnp.min(jnp.where(p2 == m2, lane, jnp.float32(1e9)), axis=1,
                 keepdims=True)
    den = m1 + m2
    m_ref[:, 0:1] = i1
    m_ref[:, 1:2] = i2
    m_ref[:, 2:3] = m1 / den
    m_ref[:, 3:4] = m2 / den

    sgm = jax.lax.dot_general(xb, sg_ref[...], (((1,), (1,)), ((), ())),
                              preferred_element_type=jnp.float32)
    sup = jax.lax.dot_general(xb, su_ref[...], (((1,), (1,)), ((), ())),
                              preferred_element_type=jnp.float32)
    sh = (sgm * jax.nn.sigmoid(sgm) * sup).astype(jnp.bfloat16)
    shared = jax.lax.dot_general(sh, sd_ref[...], (((1,), (1,)), ((), ())),
                                 preferred_element_type=jnp.float32)
    glog = jnp.sum(xb.astype(jnp.float32) * seg_ref[...], axis=1,
                   keepdims=True)
    sh_ref[...] = jax.nn.sigmoid(glog) * shared


def _ffn_body(beid_ref, xs_ref, egw_ref, euw_ref, edw_ref,
              egs_ref, eus_ref, eds_ref, ys_ref, gq_ref, uq_ref, dq_ref):
    D = xs_ref.shape[1]
    DFF = egw_ref.shape[1]
    b = pl.program_id(0)
    e = beid_ref[b]

    prev_e = jnp.where(b > 0, beid_ref[jnp.maximum(b - 1, 0)], -1)

    @pl.when(e != prev_e)
    def _dequant():
        for i in range(DFF // BLK):
            for j in range(D // BLK):
                ri = slice(i * BLK, (i + 1) * BLK)
                rj = slice(j * BLK, (j + 1) * BLK)
                gq_ref[ri, rj] = (egw_ref[0, ri, rj]
                                  * egs_ref[e, i, j]).astype(jnp.bfloat16)
                uq_ref[ri, rj] = (euw_ref[0, ri, rj]
                                  * eus_ref[e, i, j]).astype(jnp.bfloat16)
        for i in range(D // BLK):
            for j in range(DFF // BLK):
                ri = slice(i * BLK, (i + 1) * BLK)
                rj = slice(j * BLK, (j + 1) * BLK)
                dq_ref[ri, rj] = (edw_ref[0, ri, rj]
                                  * eds_ref[e, i, j]).astype(jnp.bfloat16)

    xb = xs_ref[...]
    gate = jax.lax.dot_general(xb, gq_ref[...], (((1,), (1,)), ((), ())),
                               preferred_element_type=jnp.float32)
    up = jax.lax.dot_general(xb, uq_ref[...], (((1,), (1,)), ((), ())),
                             preferred_element_type=jnp.float32)
    h = (gate * jax.nn.sigmoid(gate) * up).astype(jnp.bfloat16)
    ys_ref[...] = jax.lax.dot_general(h, dq_ref[...], (((1,), (1,)), ((), ())),
                                      preferred_element_type=jnp.float32)


def kernel(hidden_states, router_w, shared_gate_w, shared_up_w, shared_down_w,
           shared_expert_gate_w, eg_w, eg_s, eu_w, eu_s, ed_w, ed_s):
    bsz, seq, D = hidden_states.shape
    T = bsz * seq
    _, DFF, _ = eg_w.shape
    DSH = shared_gate_w.shape[0]
    NT = T // min(256, T)
    A_MAX = K * T + E * TB
    NB = A_MAX // TB

    x = hidden_states.reshape(T, D)
    x16 = x.astype(jnp.bfloat16)
    rw16 = router_w.astype(jnp.bfloat16)
    sg16 = shared_gate_w.astype(jnp.bfloat16)
    su16 = shared_up_w.astype(jnp.bfloat16)
    sd16 = shared_down_w.astype(jnp.bfloat16)
    seg16 = shared_expert_gate_w.astype(jnp.bfloat16)

    TBA = T // NT
    sh_out, m = pl.pallas_call(
        _router_shared_body,
        grid=(NT,),
        in_specs=[
            pl.BlockSpec((TBA, D), lambda t: (t, 0)),
            pl.BlockSpec((E, D), lambda t: (0, 0)),
            pl.BlockSpec((DSH, D), lambda t: (0, 0)),
            pl.BlockSpec((DSH, D), lambda t: (0, 0)),
            pl.BlockSpec((D, DSH), lambda t: (0, 0)),
            pl.BlockSpec((1, D), lambda t: (0, 0)),
        ],
        out_specs=[
            pl.BlockSpec((TBA, D), lambda t: (t, 0)),
            pl.BlockSpec((TBA, 8), lambda t: (t, 0)),
        ],
        out_shape=[
            jax.ShapeDtypeStruct((T, D), jnp.float32),
            jax.ShapeDtypeStruct((T, 8), jnp.float32),
        ],
    )(x16, rw16, sg16, su16, sd16, seg16)

    i1 = m[:, 0].astype(jnp.int32)
    i2 = m[:, 1].astype(jnp.int32)
    c1 = m[:, 2]
    c2 = m[:, 3]
    eid = jnp.stack([i1, i2], axis=1).reshape(-1)
    oh = (eid[:, None] == jnp.arange(E)[None, :]).astype(jnp.int32)
    pos_incl = jnp.cumsum(oh, axis=0)
    pos = jnp.take_along_axis(pos_incl, eid[:, None], axis=1)[:, 0] - 1
    counts = pos_incl[-1]
    nblk = (counts + TB - 1) // TB
    cumblk = jnp.cumsum(nblk)
    blk_start = cumblk - nblk
    slot = blk_start[eid] * TB + pos
    bidx = jnp.arange(NB, dtype=jnp.int32)
    beid_raw = (bidx[:, None] >= cumblk[None, :]).sum(axis=1).astype(jnp.int32)
    beid = jnp.minimum(beid_raw, E - 1)

    tid = jnp.arange(K * T, dtype=jnp.int32) // K
    xs = jnp.zeros((A_MAX, D), jnp.bfloat16).at[slot].set(x16[tid])

    ys = pl.pallas_call(
        _ffn_body,
        grid_spec=pltpu.PrefetchScalarGridSpec(
            num_scalar_prefetch=1,
            grid=(NB,),
            in_specs=[
                pl.BlockSpec((TB, D), lambda b, beid: (b, 0)),
                pl.BlockSpec((1, DFF, D), lambda b, beid: (beid[b], 0, 0)),
                pl.BlockSpec((1, DFF, D), lambda b, beid: (beid[b], 0, 0)),
                pl.BlockSpec((1, D, DFF), lambda b, beid: (beid[b], 0, 0)),
                pl.BlockSpec(memory_space=pltpu.SMEM),
                pl.BlockSpec(memory_space=pltpu.SMEM),
                pl.BlockSpec(memory_space=pltpu.SMEM),
            ],
            out_specs=pl.BlockSpec((TB, D), lambda b, beid: (b, 0)),
            scratch_shapes=[
                pltpu.VMEM((DFF, D), jnp.bfloat16),
                pltpu.VMEM((DFF, D), jnp.bfloat16),
                pltpu.VMEM((D, DFF), jnp.bfloat16),
            ],
        ),
        out_shape=jax.ShapeDtypeStruct((A_MAX, D), jnp.float32),
    )(beid, xs, eg_w, eu_w, ed_w, eg_s, eu_s, ed_s)

    s_tk = slot.reshape(T, K)
    out = (sh_out + c1[:, None] * ys[s_tk[:, 0]]
           + c2[:, None] * ys[s_tk[:, 1]])
    return out.reshape(bsz, seq, D)


# speedup vs baseline: 1.0650x; 1.0650x over previous
import functools

import jax
import jax.numpy as jnp
from jax.experimental import pallas as pl
from jax.experimental.pallas import tpu as pltpu

BLK = 128
TB = 256
E = 8
K = 2


def _router_shared_body(x16_ref, rw_ref, sg_ref, su_ref, sd_ref, seg_ref,
                        sh_ref, m_ref):
    xb = x16_ref[...]
    logits = jax.lax.dot_general(
        xb, rw_ref[...], (((1,), (1,)), ((), ())),
        preferred_element_type=jnp.float32)
    mx = jnp.max(logits, axis=1, keepdims=True)
    ex = jnp.exp(logits - mx)
    p = ex / jnp.sum(ex, axis=1, keepdims=True)
    lane = jax.lax.broadcasted_iota(jnp.int32, p.shape, 1).astype(jnp.float32)
    m1 = jnp.max(p, axis=1, keepdims=True)
    i1 = jnp.min(jnp.where(p == m1, lane, jnp.float32(1e9)), axis=1,
                 keepdims=True)
    p2 = jnp.where(lane == i1, jnp.float32(-1e30), p)
    m2 = jnp.max(p2, axis=1, keepdims=True)
    i2 = jnp.min(jnp.where(p2 == m2, lane, jnp.float32(1e9)), axis=1,
                 keepdims=True)
    den = m1 + m2
    m_ref[:, 0:1] = i1
    m_ref[:, 1:2] = i2
    m_ref[:, 2:3] = m1 / den
    m_ref[:, 3:4] = m2 / den

    sgm = jax.lax.dot_general(xb, sg_ref[...], (((1,), (1,)), ((), ())),
                              preferred_element_type=jnp.float32)
    sup = jax.lax.dot_general(xb, su_ref[...], (((1,), (1,)), ((), ())),
                              preferred_element_type=jnp.float32)
    sh = (sgm * jax.nn.sigmoid(sgm) * sup).astype(jnp.bfloat16)
    shared = jax.lax.dot_general(sh, sd_ref[...], (((1,), (1,)), ((), ())),
                                 preferred_element_type=jnp.float32)
    glog = jnp.sum(xb.astype(jnp.float32) * seg_ref[...], axis=1,
                   keepdims=True)
    sh_ref[...] = jax.nn.sigmoid(glog) * shared


def _ffn_body(beid_ref, bact_ref, xs_ref, egw_ref, euw_ref, edw_ref,
              egs_ref, eus_ref, eds_ref, ys_ref, gq_ref, uq_ref, dq_ref):
    D = xs_ref.shape[1]
    DFF = egw_ref.shape[1]
    b = pl.program_id(0)
    e = beid_ref[b]
    act = bact_ref[b] != 0

    prev_e = jnp.where(b > 0, beid_ref[jnp.maximum(b - 1, 0)], -1)

    @pl.when((e != prev_e) & act)
    def _dequant():
        for i in range(DFF // BLK):
            for j in range(D // BLK):
                ri = slice(i * BLK, (i + 1) * BLK)
                rj = slice(j * BLK, (j + 1) * BLK)
                gq_ref[ri, rj] = (egw_ref[0, ri, rj]
                                  * egs_ref[e, i, j]).astype(jnp.bfloat16)
                uq_ref[ri, rj] = (euw_ref[0, ri, rj]
                                  * eus_ref[e, i, j]).astype(jnp.bfloat16)
        for i in range(D // BLK):
            for j in range(DFF // BLK):
                ri = slice(i * BLK, (i + 1) * BLK)
                rj = slice(j * BLK, (j + 1) * BLK)
                dq_ref[ri, rj] = (edw_ref[0, ri, rj]
                                  * eds_ref[e, i, j]).astype(jnp.bfloat16)

    @pl.when(act)
    def _compute():
        xb = xs_ref[...]
        gate = jax.lax.dot_general(xb, gq_ref[...], (((1,), (1,)), ((), ())),
                                   preferred_element_type=jnp.float32)
        up = jax.lax.dot_general(xb, uq_ref[...], (((1,), (1,)), ((), ())),
                                 preferred_element_type=jnp.float32)
        h = (gate * jax.nn.sigmoid(gate) * up).astype(jnp.bfloat16)
        ys_ref[...] = jax.lax.dot_general(
            h, dq_ref[...], (((1,), (1,)), ((), ())),
            preferred_element_type=jnp.float32)


def kernel(hidden_states, router_w, shared_gate_w, shared_up_w, shared_down_w,
           shared_expert_gate_w, eg_w, eg_s, eu_w, eu_s, ed_w, ed_s):
    bsz, seq, D = hidden_states.shape
    T = bsz * seq
    _, DFF, _ = eg_w.shape
    DSH = shared_gate_w.shape[0]
    NT = T // min(256, T)
    A_MAX = K * T + E * TB
    NB = A_MAX // TB

    x = hidden_states.reshape(T, D)
    x16 = x.astype(jnp.bfloat16)
    rw16 = router_w.astype(jnp.bfloat16)
    sg16 = shared_gate_w.astype(jnp.bfloat16)
    su16 = shared_up_w.astype(jnp.bfloat16)
    sd16 = shared_down_w.astype(jnp.bfloat16)
    seg16 = shared_expert_gate_w.astype(jnp.bfloat16)

    TBA = T // NT
    sh_out, m = pl.pallas_call(
        _router_shared_body,
        grid=(NT,),
        in_specs=[
            pl.BlockSpec((TBA, D), lambda t: (t, 0)),
            pl.BlockSpec((E, D), lambda t: (0, 0)),
            pl.BlockSpec((DSH, D), lambda t: (0, 0)),
            pl.BlockSpec((DSH, D), lambda t: (0, 0)),
            pl.BlockSpec((D, DSH), lambda t: (0, 0)),
            pl.BlockSpec((1, D), lambda t: (0, 0)),
        ],
        out_specs=[
            pl.BlockSpec((TBA, D), lambda t: (t, 0)),
            pl.BlockSpec((TBA, 8), lambda t: (t, 0)),
        ],
        out_shape=[
            jax.ShapeDtypeStruct((T, D), jnp.float32),
            jax.ShapeDtypeStruct((T, 8), jnp.float32),
        ],
    )(x16, rw16, sg16, su16, sd16, seg16)

    i1 = m[:, 0].astype(jnp.int32)
    i2 = m[:, 1].astype(jnp.int32)
    c1 = m[:, 2]
    c2 = m[:, 3]
    eid = jnp.stack([i1, i2], axis=1).reshape(-1)
    oh = (eid[:, None] == jnp.arange(E)[None, :]).astype(jnp.int32)
    pos_incl = jnp.cumsum(oh, axis=0)
    pos = jnp.take_along_axis(pos_incl, eid[:, None], axis=1)[:, 0] - 1
    counts = pos_incl[-1]
    nblk = (counts + TB - 1) // TB
    cumblk = jnp.cumsum(nblk)
    blk_start = cumblk - nblk
    slot = blk_start[eid] * TB + pos
    bidx = jnp.arange(NB, dtype=jnp.int32)
    beid_raw = (bidx[:, None] >= cumblk[None, :]).sum(axis=1).astype(jnp.int32)
    beid = jnp.minimum(beid_raw, E - 1)
    bact = (beid_raw < E).astype(jnp.int32)

    tid = jnp.arange(K * T, dtype=jnp.int32) // K
    tid_of_slot = jnp.zeros((A_MAX,), jnp.int32).at[slot].set(tid)
    xs = x[tid_of_slot].astype(jnp.bfloat16)

    ys = pl.pallas_call(
        _ffn_body,
        grid_spec=pltpu.PrefetchScalarGridSpec(
            num_scalar_prefetch=2,
            grid=(NB,),
            in_specs=[
                pl.BlockSpec((TB, D), lambda b, beid, bact: (b, 0)),
                pl.BlockSpec((1, DFF, D),
                             lambda b, beid, bact: (beid[b], 0, 0)),
                pl.BlockSpec((1, DFF, D),
                             lambda b, beid, bact: (beid[b], 0, 0)),
                pl.BlockSpec((1, D, DFF),
                             lambda b, beid, bact: (beid[b], 0, 0)),
                pl.BlockSpec(memory_space=pltpu.SMEM),
                pl.BlockSpec(memory_space=pltpu.SMEM),
                pl.BlockSpec(memory_space=pltpu.SMEM),
            ],
            out_specs=pl.BlockSpec((TB, D), lambda b, beid, bact: (b, 0)),
            scratch_shapes=[
                pltpu.VMEM((DFF, D), jnp.bfloat16),
                pltpu.VMEM((DFF, D), jnp.bfloat16),
                pltpu.VMEM((D, DFF), jnp.bfloat16),
            ],
        ),
        out_shape=jax.ShapeDtypeStruct((A_MAX, D), jnp.float32),
    )(beid, bact, xs, eg_w, eu_w, ed_w, eg_s, eu_s, ed_s)

    s_tk = slot.reshape(T, K)
    out = (sh_out + c1[:, None] * ys[s_tk[:, 0]]
           + c2[:, None] * ys[s_tk[:, 1]])
    return out.reshape(bsz, seq, D)
